# baseline (speedup 1.0000x reference)
import dataclasses
import functools

import jax
import jax.numpy as jnp
from jax import lax
from jax.experimental import pallas as pl
from jax.experimental.pallas import tpu as pltpu
from jax.experimental.pallas import tpu_sc as plsc

_N = 10000
_NP = 10240
_E = 320000
_D = 128
_Z = 64
_P = 65536
_BLK = 256
_BB = 512

_NC = 2
_NS = 16
_NW = _NC * _NS
_EC = 128
_NCH = 80
_EP = _NW * _NCH * _EC
_K0 = 114
_K1 = 46
_RW = _NP // _NS
_PW = 2 * _P // _NW

_sc_mesh = plsc.VectorSubcoreMesh(core_axis_name="c", subcore_axis_name="s")

_sc_params = pltpu.CompilerParams()
if "needs_layout_passes" in pltpu.CompilerParams.__dataclass_fields__:
    _sc_params = dataclasses.replace(_sc_params, needs_layout_passes=False)


def _scale_rows(rows, ebuf):
    @pl.loop(0, _EC)
    def _(r):
        ridx = jnp.full((16,), r, jnp.int32)
        vidx = jnp.full((16,), 2, jnp.int32)
        val = plsc.bitcast(plsc.load_gather(ebuf, [vidx, ridx]), jnp.float32)
        for g in range(_D // 16):
            sl = (r, pl.ds(g * 16, 16))
            rows[sl] = rows[sl] * val


def _spmm_body(x_hbm, ed_hbm, out_hbm,
               eb0_v, eb1_v, rows0_v, rows1_v, acc_sh,
               gsem0, gsem1, esem0, esem1):
    c = lax.axis_index("c")
    s = lax.axis_index("s")
    wid = s * _NC + c
    rows = (rows0_v, rows1_v)
    ebuf = (eb0_v, eb1_v)
    gsem = (gsem0, gsem1)
    esem = (esem0, esem1)

    with jax.named_scope("zero_acc"):
        @pl.loop(0, _EC)
        def _(i):
            for g in range(_D // 16):
                rows0_v[i, pl.ds(g * 16, 16)] = jnp.zeros((16,), jnp.float32)

        @pl.loop(0, _RW // _EC)
        def _(k):
            pltpu.sync_copy(rows0_v, acc_sh.at[pl.ds(s * _RW + k * _EC, _EC)])

    base = lax.select(c == 0, s * _K0, _NS * _K0 + s * _K1)
    cnt = lax.select(c == 0, _K0, _K1)

    pltpu.async_copy(ed_hbm.at[base], eb0_v, esem0)
    pltpu.make_async_copy(ed_hbm.at[base], eb0_v, esem0).wait()
    pltpu.async_copy(x_hbm.at[eb0_v.at[0]], rows0_v, gsem0)
    pltpu.async_copy(ed_hbm.at[base + 1], eb1_v, esem1)
    plsc.subcore_barrier()

    with jax.named_scope("edge_loop"):
        @pl.loop(0, cnt, step=2)
        def _(k0):
          for b in range(2):
            k = k0 + b
            pltpu.make_async_copy(x_hbm.at[ebuf[b].at[0]], rows[b],
                                  gsem[b]).wait()

            @pl.when(k + 1 < cnt)
            def _():
                pltpu.make_async_copy(ed_hbm.at[base + k + 1], ebuf[1 - b],
                                      esem[1 - b]).wait()
                pltpu.async_copy(x_hbm.at[ebuf[1 - b].at[0]], rows[1 - b],
                                 gsem[1 - b])

            _scale_rows(rows[b], ebuf[b])
            pltpu.sync_copy(rows[b], acc_sh.at[ebuf[b].at[1]], add=True)

            @pl.when(k + 2 < cnt)
            def _():
                pltpu.async_copy(ed_hbm.at[base + k + 2], ebuf[b], esem[b])

    with jax.named_scope("tail_barrier"):
        plsc.subcore_barrier()
    with jax.named_scope("copy_out"):
        pltpu.sync_copy(acc_sh.at[pl.ds(s * _RW, _RW)],
                        out_hbm.at[c, pl.ds(s * _RW, _RW)])


def _spmm_partials(x, edata):
    k = pl.kernel(
        _spmm_body,
        out_type=jax.ShapeDtypeStruct((_NC, _NP, _D), jnp.float32),
        mesh=_sc_mesh,
        scratch_types=[
            pltpu.VMEM((3, _EC), jnp.int32),
            pltpu.VMEM((3, _EC), jnp.int32),
            pltpu.VMEM((_EC, _D), jnp.float32),
            pltpu.VMEM((_EC, _D), jnp.float32),
            pltpu.VMEM_SHARED((_NP, _D), jnp.float32),
            pltpu.SemaphoreType.DMA,
            pltpu.SemaphoreType.DMA,
            pltpu.SemaphoreType.DMA,
            pltpu.SemaphoreType.DMA,
        ],
        compiler_params=_sc_params,
    )
    return k(x, edata)


def _pairs_body(lat_hbm, idx_hbm, out_hbm, idx_v, rows_v):
    c = lax.axis_index("c")
    s = lax.axis_index("s")
    wid = s * _NC + c
    base = wid * _PW

    @pl.loop(0, _PW // 128)
    def _(k):
        b = base + k * 128
        pltpu.sync_copy(idx_hbm.at[pl.ds(b, 128)], idx_v)
        pltpu.sync_copy(lat_hbm.at[idx_v], rows_v)
        pltpu.sync_copy(rows_v, out_hbm.at[pl.ds(b, 128)])


def _pair_gather(latent_p, idxflat):
    k = pl.kernel(
        _pairs_body,
        out_type=jax.ShapeDtypeStruct((2 * _P, _D), jnp.float32),
        mesh=_sc_mesh,
        scratch_types=[
            pltpu.VMEM((128,), jnp.int32),
            pltpu.VMEM((128, _D), jnp.float32),
        ],
    )
    return k(latent_p, idxflat)


def _mm_body(x_ref, w_ref, o_ref):
    o_ref[...] = jnp.dot(x_ref[...], w_ref[...],
                         preferred_element_type=jnp.float32)


def _matmul(x, w):
    return pl.pallas_call(
        _mm_body,
        grid=(_NP // _BLK,),
        in_specs=[pl.BlockSpec((_BLK, _D), lambda i: (i, 0)),
                  pl.BlockSpec((_D, _D), lambda i: (0, 0))],
        out_specs=pl.BlockSpec((_BLK, _D), lambda i: (i, 0)),
        out_shape=jax.ShapeDtypeStruct((_NP, _D), jnp.float32),
    )(x, w)


def _fuse_body(p0_ref, p1_ref, b_ref, r_ref, w_ref, x_ref, h_ref):
    x = p0_ref[...] + p1_ref[...] + b_ref[...] + r_ref[...]
    x_ref[...] = x
    h_ref[...] = jnp.dot(x, w_ref[...], preferred_element_type=jnp.float32)


def _fused_layer(p, b, res, w):
    return pl.pallas_call(
        _fuse_body,
        grid=(_NP // _BLK,),
        in_specs=[pl.BlockSpec((_BLK, _D), lambda i: (i, 0)),
                  pl.BlockSpec((_BLK, _D), lambda i: (i, 0)),
                  pl.BlockSpec((1, _D), lambda i: (0, 0)),
                  pl.BlockSpec((_BLK, _D), lambda i: (i, 0)),
                  pl.BlockSpec((_D, _D), lambda i: (0, 0))],
        out_specs=[pl.BlockSpec((_BLK, _D), lambda i: (i, 0)),
                   pl.BlockSpec((_BLK, _D), lambda i: (i, 0))],
        out_shape=[jax.ShapeDtypeStruct((_NP, _D), jnp.float32),
                   jax.ShapeDtypeStruct((_NP, _D), jnp.float32)],
    )(p[0], p[1], b, res, w)


def _final_body(p0_ref, p1_ref, b_ref, r_ref, o_ref):
    o_ref[...] = p0_ref[...] + p1_ref[...] + b_ref[...] + r_ref[...]


def _final_layer(p, b, res):
    return pl.pallas_call(
        _final_body,
        grid=(_NP // _BLK,),
        in_specs=[pl.BlockSpec((_BLK, _D), lambda i: (i, 0)),
                  pl.BlockSpec((_BLK, _D), lambda i: (i, 0)),
                  pl.BlockSpec((1, _D), lambda i: (0, 0)),
                  pl.BlockSpec((_BLK, _D), lambda i: (i, 0))],
        out_specs=pl.BlockSpec((_BLK, _D), lambda i: (i, 0)),
        out_shape=jax.ShapeDtypeStruct((_NP, _D), jnp.float32),
    )(p[0], p[1], b, res)


def _bilinear_body(fa_ref, fb_ref, wbr_ref, ones_ref, bb_ref, wd1_ref,
                   bd1_ref, wd2_ref, bd2_ref, out_ref):
    fa = fa_ref[...]
    fb = fb_ref[...].astype(jnp.bfloat16)
    t = jnp.dot(fa.astype(jnp.bfloat16), wbr_ref[...],
                preferred_element_type=jnp.float32)
    g = t.astype(jnp.bfloat16).reshape(_BB, _Z, _D) * fb[:, None, :]
    g = g.reshape(_BB, _Z * _D)
    feat = jnp.dot(g, ones_ref[...], preferred_element_type=jnp.float32)
    feat = feat + bb_ref[...]
    feat = jnp.where(feat > 0, feat, jnp.exp(jnp.minimum(feat, 0.0)) - 1.0)
    h = jnp.dot(feat, wd1_ref[...], preferred_element_type=jnp.float32)
    h = h + bd1_ref[...]
    h = jnp.where(h > 0, h, jnp.exp(jnp.minimum(h, 0.0)) - 1.0)
    pred = jnp.dot(h, wd2_ref[...], preferred_element_type=jnp.float32)
    out_ref[...] = pred + bd2_ref[...]


def _bilinear_decode(fab, Wb, bb, Wd1, bd1, Wd2, bd2):
    wbr = Wb.transpose(1, 0, 2).reshape(_D, _Z * _D).astype(jnp.bfloat16)
    ones_bd = (jnp.arange(_Z * _D, dtype=jnp.int32)[:, None] // _D
               == jnp.arange(_Z, dtype=jnp.int32)[None, :]).astype(jnp.bfloat16)
    nblk = _P // _BB
    return pl.pallas_call(
        _bilinear_body,
        grid=(nblk,),
        in_specs=[
            pl.BlockSpec((_BB, _D), lambda i: (i, 0)),
            pl.BlockSpec((_BB, _D), lambda i: (i + _P // _BB, 0)),
            pl.BlockSpec((_D, _Z * _D), lambda i: (0, 0)),
            pl.BlockSpec((_Z * _D, _Z), lambda i: (0, 0)),
            pl.BlockSpec((_Z,), lambda i: (0,)),
            pl.BlockSpec((_Z, _Z), lambda i: (0, 0)),
            pl.BlockSpec((_Z,), lambda i: (0,)),
            pl.BlockSpec((_Z, 1), lambda i: (0, 0)),
            pl.BlockSpec((1,), lambda i: (0,)),
        ],
        out_specs=pl.BlockSpec((_BB, 1), lambda i: (i, 0)),
        out_shape=jax.ShapeDtypeStruct((_P, 1), jnp.float32),
    )(fab, fab, wbr, ones_bd, bb, Wd1, bd1, Wd2, bd2)


def kernel(adj_indices, adj_values, features, idx, W0, b0, W1, b1, W2, b2,
           Wb, bb, Wd1, bd1, Wd2, bd2):
    pad_e = _EP - _E
    src3 = jnp.pad(adj_indices[0], (0, pad_e)).reshape(_NW, _NCH, 1, _EC)
    dst3 = jnp.pad(adj_indices[1], (0, pad_e)).reshape(_NW, _NCH, 1, _EC)
    vals3 = lax.bitcast_convert_type(
        jnp.pad(adj_values, (0, pad_e)), jnp.int32
    ).reshape(_NW, _NCH, 1, _EC)
    edata = jnp.concatenate([src3, dst3, vals3],
                            axis=2).reshape(_NW * _NCH, 3, _EC)
    feats_p = jnp.pad(features, ((0, _NP - _N), (0, 0)))
    b0r = b0.reshape(1, _D)
    b1r = b1.reshape(1, _D)
    b2r = b2.reshape(1, _D)

    h0 = _matmul(feats_p, W0)
    p0 = _spmm_partials(h0, edata)
    x1, h1 = _fused_layer(p0, b0r, jnp.zeros_like(h0), W1)
    p1 = _spmm_partials(h1, edata)
    x2, h2 = _fused_layer(p1, b1r, x1, W2)
    p2 = _spmm_partials(h2, edata)
    latent_p = _final_layer(p2, b2r, x2)

    idxflat = jnp.concatenate([idx[0], idx[1]])
    fab = _pair_gather(latent_p, idxflat)
    pred = _bilinear_decode(fab, Wb, bb, Wd1, bd1, Wd2, bd2)
    return (pred, latent_p[:_N])


# speedup vs baseline: 1.3691x; 1.3691x over previous
import dataclasses
import functools

import jax
import jax.numpy as jnp
from jax import lax
from jax.experimental import pallas as pl
from jax.experimental.pallas import tpu as pltpu
from jax.experimental.pallas import tpu_sc as plsc

_N = 10000
_NP = 10240
_E = 320000
_D = 128
_Z = 64
_P = 65536
_BLK = 256
_BB = 512

_NC = 2
_NS = 16
_NW = _NC * _NS
_EC = 128
_NCH = 80
_EP = _NW * _NCH * _EC
_K0 = 80
_K1 = 80
_RW = _NP // _NS
_PW = 2 * _P // _NW

_sc_mesh = plsc.VectorSubcoreMesh(core_axis_name="c", subcore_axis_name="s")

_sc_params = pltpu.CompilerParams()
if "needs_layout_passes" in pltpu.CompilerParams.__dataclass_fields__:
    _sc_params = dataclasses.replace(_sc_params, needs_layout_passes=False)


def _scale_rows(rows, ebuf):
    @pl.loop(0, _EC)
    def _(r):
        ridx = jnp.full((16,), r, jnp.int32)
        vidx = jnp.full((16,), 2, jnp.int32)
        val = plsc.bitcast(plsc.load_gather(ebuf, [vidx, ridx]), jnp.float32)
        for g in range(_D // 16):
            sl = (r, pl.ds(g * 16, 16))
            rows[sl] = rows[sl] * val


def _spmm_body(x_hbm, ed_hbm, out_hbm,
               eb0_v, eb1_v, rows0_v, rows1_v, acc_sh,
               gsem0, gsem1, esem0, esem1):
    c = lax.axis_index("c")
    s = lax.axis_index("s")
    wid = s * _NC + c
    rows = (rows0_v, rows1_v)
    ebuf = (eb0_v, eb1_v)
    gsem = (gsem0, gsem1)
    esem = (esem0, esem1)

    with jax.named_scope("zero_acc"):
        @pl.loop(0, _EC)
        def _(i):
            for g in range(_D // 16):
                rows0_v[i, pl.ds(g * 16, 16)] = jnp.zeros((16,), jnp.float32)

        @pl.loop(0, _RW // _EC)
        def _(k):
            pltpu.sync_copy(rows0_v, acc_sh.at[pl.ds(s * _RW + k * _EC, _EC)])

    base = lax.select(c == 0, s * _K0, _NS * _K0 + s * _K1)
    cnt = lax.select(c == 0, _K0, _K1)

    pltpu.async_copy(ed_hbm.at[base], eb0_v, esem0)
    pltpu.make_async_copy(ed_hbm.at[base], eb0_v, esem0).wait()
    pltpu.async_copy(x_hbm.at[eb0_v.at[0]], rows0_v, gsem0)
    pltpu.async_copy(ed_hbm.at[base + 1], eb1_v, esem1)
    plsc.subcore_barrier()

    with jax.named_scope("edge_loop"):
        @pl.loop(0, cnt, step=2)
        def _(k0):
          for b in range(2):
            k = k0 + b
            pltpu.make_async_copy(x_hbm.at[ebuf[b].at[0]], rows[b],
                                  gsem[b]).wait()

            @pl.when(k + 1 < cnt)
            def _():
                pltpu.make_async_copy(ed_hbm.at[base + k + 1], ebuf[1 - b],
                                      esem[1 - b]).wait()
                pltpu.async_copy(x_hbm.at[ebuf[1 - b].at[0]], rows[1 - b],
                                 gsem[1 - b])

            _scale_rows(rows[b], ebuf[b])
            pltpu.sync_copy(rows[b], acc_sh.at[ebuf[b].at[1]], add=True)

            @pl.when(k + 2 < cnt)
            def _():
                pltpu.async_copy(ed_hbm.at[base + k + 2], ebuf[b], esem[b])

    with jax.named_scope("tail_barrier"):
        plsc.subcore_barrier()
    with jax.named_scope("copy_out"):
        pltpu.sync_copy(acc_sh.at[pl.ds(s * _RW, _RW)],
                        out_hbm.at[c, pl.ds(s * _RW, _RW)])


def _spmm_partials(x, edata):
    k = pl.kernel(
        _spmm_body,
        out_type=jax.ShapeDtypeStruct((_NC, _NP, _D), jnp.float32),
        mesh=_sc_mesh,
        scratch_types=[
            pltpu.VMEM((3, _EC), jnp.int32),
            pltpu.VMEM((3, _EC), jnp.int32),
            pltpu.VMEM((_EC, _D), jnp.float32),
            pltpu.VMEM((_EC, _D), jnp.float32),
            pltpu.VMEM_SHARED((_NP, _D), jnp.float32),
            pltpu.SemaphoreType.DMA,
            pltpu.SemaphoreType.DMA,
            pltpu.SemaphoreType.DMA,
            pltpu.SemaphoreType.DMA,
        ],
        compiler_params=_sc_params,
    )
    return k(x, edata)


def _pairs_body(lat_hbm, idx_hbm, out_hbm, idx_v, rows_v):
    c = lax.axis_index("c")
    s = lax.axis_index("s")
    wid = s * _NC + c
    base = wid * _PW

    @pl.loop(0, _PW // 128)
    def _(k):
        b = base + k * 128
        pltpu.sync_copy(idx_hbm.at[pl.ds(b, 128)], idx_v)
        pltpu.sync_copy(lat_hbm.at[idx_v], rows_v)
        pltpu.sync_copy(rows_v, out_hbm.at[pl.ds(b, 128)])


def _pair_gather(latent_p, idxflat):
    k = pl.kernel(
        _pairs_body,
        out_type=jax.ShapeDtypeStruct((2 * _P, _D), jnp.float32),
        mesh=_sc_mesh,
        scratch_types=[
            pltpu.VMEM((128,), jnp.int32),
            pltpu.VMEM((128, _D), jnp.float32),
        ],
    )
    return k(latent_p, idxflat)


def _mm_body(x_ref, w_ref, o_ref):
    o_ref[...] = jnp.dot(x_ref[...], w_ref[...],
                         preferred_element_type=jnp.float32)


def _matmul(x, w):
    return pl.pallas_call(
        _mm_body,
        grid=(_NP // _BLK,),
        in_specs=[pl.BlockSpec((_BLK, _D), lambda i: (i, 0)),
                  pl.BlockSpec((_D, _D), lambda i: (0, 0))],
        out_specs=pl.BlockSpec((_BLK, _D), lambda i: (i, 0)),
        out_shape=jax.ShapeDtypeStruct((_NP, _D), jnp.float32),
    )(x, w)


def _fuse_body(p0_ref, p1_ref, b_ref, r_ref, w_ref, x_ref, h_ref):
    x = p0_ref[...] + p1_ref[...] + b_ref[...] + r_ref[...]
    x_ref[...] = x
    h_ref[...] = jnp.dot(x, w_ref[...], preferred_element_type=jnp.float32)


def _fused_layer(p, b, res, w):
    return pl.pallas_call(
        _fuse_body,
        grid=(_NP // _BLK,),
        in_specs=[pl.BlockSpec((_BLK, _D), lambda i: (i, 0)),
                  pl.BlockSpec((_BLK, _D), lambda i: (i, 0)),
                  pl.BlockSpec((1, _D), lambda i: (0, 0)),
                  pl.BlockSpec((_BLK, _D), lambda i: (i, 0)),
                  pl.BlockSpec((_D, _D), lambda i: (0, 0))],
        out_specs=[pl.BlockSpec((_BLK, _D), lambda i: (i, 0)),
                   pl.BlockSpec((_BLK, _D), lambda i: (i, 0))],
        out_shape=[jax.ShapeDtypeStruct((_NP, _D), jnp.float32),
                   jax.ShapeDtypeStruct((_NP, _D), jnp.float32)],
    )(p[0], p[1], b, res, w)


def _final_body(p0_ref, p1_ref, b_ref, r_ref, o_ref):
    o_ref[...] = p0_ref[...] + p1_ref[...] + b_ref[...] + r_ref[...]


def _final_layer(p, b, res):
    return pl.pallas_call(
        _final_body,
        grid=(_NP // _BLK,),
        in_specs=[pl.BlockSpec((_BLK, _D), lambda i: (i, 0)),
                  pl.BlockSpec((_BLK, _D), lambda i: (i, 0)),
                  pl.BlockSpec((1, _D), lambda i: (0, 0)),
                  pl.BlockSpec((_BLK, _D), lambda i: (i, 0))],
        out_specs=pl.BlockSpec((_BLK, _D), lambda i: (i, 0)),
        out_shape=jax.ShapeDtypeStruct((_NP, _D), jnp.float32),
    )(p[0], p[1], b, res)


def _bilinear_body(fa_ref, fb_ref, wbr_ref, ones_ref, bb_ref, wd1_ref,
                   bd1_ref, wd2_ref, bd2_ref, out_ref):
    fa = fa_ref[...]
    fb = fb_ref[...].astype(jnp.bfloat16)
    t = jnp.dot(fa.astype(jnp.bfloat16), wbr_ref[...],
                preferred_element_type=jnp.float32)
    g = t.astype(jnp.bfloat16).reshape(_BB, _Z, _D) * fb[:, None, :]
    g = g.reshape(_BB, _Z * _D)
    feat = jnp.dot(g, ones_ref[...], preferred_element_type=jnp.float32)
    feat = feat + bb_ref[...]
    feat = jnp.where(feat > 0, feat, jnp.exp(jnp.minimum(feat, 0.0)) - 1.0)
    h = jnp.dot(feat, wd1_ref[...], preferred_element_type=jnp.float32)
    h = h + bd1_ref[...]
    h = jnp.where(h > 0, h, jnp.exp(jnp.minimum(h, 0.0)) - 1.0)
    pred = jnp.dot(h, wd2_ref[...], preferred_element_type=jnp.float32)
    out_ref[...] = pred + bd2_ref[...]


def _bilinear_decode(fab, Wb, bb, Wd1, bd1, Wd2, bd2):
    wbr = Wb.transpose(1, 0, 2).reshape(_D, _Z * _D).astype(jnp.bfloat16)
    ones_bd = (jnp.arange(_Z * _D, dtype=jnp.int32)[:, None] // _D
               == jnp.arange(_Z, dtype=jnp.int32)[None, :]).astype(jnp.bfloat16)
    nblk = _P // _BB
    return pl.pallas_call(
        _bilinear_body,
        grid=(nblk,),
        in_specs=[
            pl.BlockSpec((_BB, _D), lambda i: (i, 0)),
            pl.BlockSpec((_BB, _D), lambda i: (i + _P // _BB, 0)),
            pl.BlockSpec((_D, _Z * _D), lambda i: (0, 0)),
            pl.BlockSpec((_Z * _D, _Z), lambda i: (0, 0)),
            pl.BlockSpec((_Z,), lambda i: (0,)),
            pl.BlockSpec((_Z, _Z), lambda i: (0, 0)),
            pl.BlockSpec((_Z,), lambda i: (0,)),
            pl.BlockSpec((_Z, 1), lambda i: (0, 0)),
            pl.BlockSpec((1,), lambda i: (0,)),
        ],
        out_specs=pl.BlockSpec((_BB, 1), lambda i: (i, 0)),
        out_shape=jax.ShapeDtypeStruct((_P, 1), jnp.float32),
    )(fab, fab, wbr, ones_bd, bb, Wd1, bd1, Wd2, bd2)


def kernel(adj_indices, adj_values, features, idx, W0, b0, W1, b1, W2, b2,
           Wb, bb, Wd1, bd1, Wd2, bd2):
    pad_e = _EP - _E
    spread = jnp.arange(pad_e, dtype=jnp.int32)
    src3 = jnp.concatenate([adj_indices[0], spread]).reshape(
        _NW, _NCH, 1, _EC)
    dst3 = jnp.concatenate([adj_indices[1], spread]).reshape(
        _NW, _NCH, 1, _EC)
    vals3 = lax.bitcast_convert_type(
        jnp.pad(adj_values, (0, pad_e)), jnp.int32
    ).reshape(_NW, _NCH, 1, _EC)
    edata = jnp.concatenate([src3, dst3, vals3],
                            axis=2).reshape(_NW * _NCH, 3, _EC)
    feats_p = jnp.pad(features, ((0, _NP - _N), (0, 0)))
    b0r = b0.reshape(1, _D)
    b1r = b1.reshape(1, _D)
    b2r = b2.reshape(1, _D)

    h0 = _matmul(feats_p, W0)
    p0 = _spmm_partials(h0, edata)
    x1, h1 = _fused_layer(p0, b0r, jnp.zeros_like(h0), W1)
    p1 = _spmm_partials(h1, edata)
    x2, h2 = _fused_layer(p1, b1r, x1, W2)
    p2 = _spmm_partials(h2, edata)
    latent_p = _final_layer(p2, b2r, x2)

    idxflat = jnp.concatenate([idx[0], idx[1]])
    fab = _pair_gather(latent_p, idxflat)
    pred = _bilinear_decode(fab, Wb, bb, Wd1, bd1, Wd2, bd2)
    return (pred, latent_p[:_N])


# speedup vs baseline: 1.7772x; 1.2980x over previous
import dataclasses
import functools

import jax
import jax.numpy as jnp
from jax import lax
from jax.experimental import pallas as pl
from jax.experimental.pallas import tpu as pltpu
from jax.experimental.pallas import tpu_sc as plsc

_N = 10000
_NP = 10240
_E = 320000
_D = 128
_Z = 64
_P = 65536
_BLK = 256
_BB = 512

_NC = 2
_NS = 16
_NW = _NC * _NS
_EC = 128
_NCH = 80
_EP = _NW * _NCH * _EC
_K0 = 80
_K1 = 80
_RW = _NP // _NS
_PW = 2 * _P // _NW

_sc_mesh = plsc.VectorSubcoreMesh(core_axis_name="c", subcore_axis_name="s")

_sc_params = pltpu.CompilerParams()
if "needs_layout_passes" in pltpu.CompilerParams.__dataclass_fields__:
    _sc_params = dataclasses.replace(_sc_params, needs_layout_passes=False)


def _scale_rows(rows, ebuf):
    @pl.loop(0, _EC)
    def _(r):
        ridx = jnp.full((16,), r, jnp.int32)
        vidx = jnp.full((16,), 2, jnp.int32)
        val = plsc.bitcast(plsc.load_gather(ebuf, [vidx, ridx]), jnp.float32)
        for g in range(_D // 16):
            sl = (r, pl.ds(g * 16, 16))
            rows[sl] = rows[sl] * val


def _spmm_body(x_hbm, ed_hbm, out_hbm,
               eb0_v, eb1_v, rows0_v, rows1_v, acc_sh,
               gsem0, gsem1, esem0, esem1):
    c = lax.axis_index("c")
    s = lax.axis_index("s")
    wid = s * _NC + c
    rows = (rows0_v, rows1_v)
    ebuf = (eb0_v, eb1_v)
    gsem = (gsem0, gsem1)
    esem = (esem0, esem1)

    with jax.named_scope("zero_acc"):
        @pl.loop(0, _EC)
        def _(i):
            for g in range(_D // 16):
                rows0_v[i, pl.ds(g * 16, 16)] = jnp.zeros((16,), jnp.float32)

        @pl.loop(0, _RW // _EC)
        def _(k):
            pltpu.sync_copy(rows0_v, acc_sh.at[pl.ds(s * _RW + k * _EC, _EC)])

    base = lax.select(c == 0, s * _K0, _NS * _K0 + s * _K1)
    cnt = lax.select(c == 0, _K0, _K1)

    pltpu.async_copy(ed_hbm.at[base], eb0_v, esem0)
    pltpu.make_async_copy(ed_hbm.at[base], eb0_v, esem0).wait()
    pltpu.async_copy(x_hbm.at[eb0_v.at[0]], rows0_v, gsem0)
    pltpu.async_copy(ed_hbm.at[base + 1], eb1_v, esem1)
    plsc.subcore_barrier()

    with jax.named_scope("edge_loop"):
        @pl.loop(0, cnt, step=2)
        def _(k0):
          for b in range(2):
            k = k0 + b
            pltpu.make_async_copy(x_hbm.at[ebuf[b].at[0]], rows[b],
                                  gsem[b]).wait()

            @pl.when(k + 1 < cnt)
            def _():
                pltpu.make_async_copy(ed_hbm.at[base + k + 1], ebuf[1 - b],
                                      esem[1 - b]).wait()
                pltpu.async_copy(x_hbm.at[ebuf[1 - b].at[0]], rows[1 - b],
                                 gsem[1 - b])

            _scale_rows(rows[b], ebuf[b])
            pltpu.sync_copy(rows[b], acc_sh.at[ebuf[b].at[1]], add=True)

            @pl.when(k + 2 < cnt)
            def _():
                pltpu.async_copy(ed_hbm.at[base + k + 2], ebuf[b], esem[b])

    with jax.named_scope("tail_barrier"):
        plsc.subcore_barrier()
    with jax.named_scope("copy_out"):
        pltpu.sync_copy(acc_sh.at[pl.ds(s * _RW, _RW)],
                        out_hbm.at[c, pl.ds(s * _RW, _RW)])


def _spmm_partials(x, edata):
    k = pl.kernel(
        _spmm_body,
        out_type=jax.ShapeDtypeStruct((_NC, _NP, _D), jnp.float32),
        mesh=_sc_mesh,
        scratch_types=[
            pltpu.VMEM((3, _EC), jnp.int32),
            pltpu.VMEM((3, _EC), jnp.int32),
            pltpu.VMEM((_EC, _D), jnp.float32),
            pltpu.VMEM((_EC, _D), jnp.float32),
            pltpu.VMEM_SHARED((_NP, _D), jnp.float32),
            pltpu.SemaphoreType.DMA,
            pltpu.SemaphoreType.DMA,
            pltpu.SemaphoreType.DMA,
            pltpu.SemaphoreType.DMA,
        ],
        compiler_params=_sc_params,
    )
    return k(x, edata)


def _pairs_body(lat_hbm, idx_hbm, out_hbm, idx_v, rows_v):
    c = lax.axis_index("c")
    s = lax.axis_index("s")
    wid = s * _NC + c
    base = wid * _PW

    @pl.loop(0, _PW // 128)
    def _(k):
        b = base + k * 128
        pltpu.sync_copy(idx_hbm.at[pl.ds(b, 128)], idx_v)
        pltpu.sync_copy(lat_hbm.at[idx_v], rows_v)
        pltpu.sync_copy(rows_v, out_hbm.at[pl.ds(b, 128)])


def _pair_gather(latent_p, idxflat):
    k = pl.kernel(
        _pairs_body,
        out_type=jax.ShapeDtypeStruct((2 * _P, _D), jnp.float32),
        mesh=_sc_mesh,
        scratch_types=[
            pltpu.VMEM((128,), jnp.int32),
            pltpu.VMEM((128, _D), jnp.float32),
        ],
    )
    return k(latent_p, idxflat)


def _mm_body(x_ref, w_ref, o_ref):
    o_ref[...] = jnp.dot(x_ref[...], w_ref[...],
                         preferred_element_type=jnp.float32)


def _matmul(x, w):
    return pl.pallas_call(
        _mm_body,
        grid=(_NP // _BLK,),
        in_specs=[pl.BlockSpec((_BLK, _D), lambda i: (i, 0)),
                  pl.BlockSpec((_D, _D), lambda i: (0, 0))],
        out_specs=pl.BlockSpec((_BLK, _D), lambda i: (i, 0)),
        out_shape=jax.ShapeDtypeStruct((_NP, _D), jnp.float32),
    )(x, w)


def _fuse_body(p0_ref, p1_ref, b_ref, r_ref, w_ref, x_ref, h_ref):
    x = p0_ref[...] + p1_ref[...] + b_ref[...] + r_ref[...]
    x_ref[...] = x
    h_ref[...] = jnp.dot(x, w_ref[...], preferred_element_type=jnp.float32)


def _fused_layer(p, b, res, w):
    return pl.pallas_call(
        _fuse_body,
        grid=(_NP // _BLK,),
        in_specs=[pl.BlockSpec((_BLK, _D), lambda i: (i, 0)),
                  pl.BlockSpec((_BLK, _D), lambda i: (i, 0)),
                  pl.BlockSpec((1, _D), lambda i: (0, 0)),
                  pl.BlockSpec((_BLK, _D), lambda i: (i, 0)),
                  pl.BlockSpec((_D, _D), lambda i: (0, 0))],
        out_specs=[pl.BlockSpec((_BLK, _D), lambda i: (i, 0)),
                   pl.BlockSpec((_BLK, _D), lambda i: (i, 0))],
        out_shape=[jax.ShapeDtypeStruct((_NP, _D), jnp.float32),
                   jax.ShapeDtypeStruct((_NP, _D), jnp.float32)],
    )(p[0], p[1], b, res, w)


def _final_body(p0_ref, p1_ref, b_ref, r_ref, o_ref):
    o_ref[...] = p0_ref[...] + p1_ref[...] + b_ref[...] + r_ref[...]


def _final_layer(p, b, res):
    return pl.pallas_call(
        _final_body,
        grid=(_NP // _BLK,),
        in_specs=[pl.BlockSpec((_BLK, _D), lambda i: (i, 0)),
                  pl.BlockSpec((_BLK, _D), lambda i: (i, 0)),
                  pl.BlockSpec((1, _D), lambda i: (0, 0)),
                  pl.BlockSpec((_BLK, _D), lambda i: (i, 0))],
        out_specs=pl.BlockSpec((_BLK, _D), lambda i: (i, 0)),
        out_shape=jax.ShapeDtypeStruct((_NP, _D), jnp.float32),
    )(p[0], p[1], b, res)


def _bilinear_body(fa_ref, fb_ref, wbr_ref, ones_ref, bb_ref, wd1_ref,
                   bd1_ref, wd2_ref, bd2_ref, out_ref):
    fa = fa_ref[...]
    fb = fb_ref[...].astype(jnp.bfloat16)
    t = jnp.dot(fa.astype(jnp.bfloat16), wbr_ref[...],
                preferred_element_type=jnp.float32)
    fbt = jnp.concatenate([fb] * _Z, axis=1)
    g = t.astype(jnp.bfloat16) * fbt
    feat = jnp.dot(g, ones_ref[...], preferred_element_type=jnp.float32)
    feat = feat + bb_ref[...]
    feat = jnp.where(feat > 0, feat, jnp.exp(jnp.minimum(feat, 0.0)) - 1.0)
    h = jnp.dot(feat, wd1_ref[...], preferred_element_type=jnp.float32)
    h = h + bd1_ref[...]
    h = jnp.where(h > 0, h, jnp.exp(jnp.minimum(h, 0.0)) - 1.0)
    pred = jnp.dot(h, wd2_ref[...], preferred_element_type=jnp.float32)
    out_ref[...] = pred + bd2_ref[...]


def _bilinear_decode(fab, Wb, bb, Wd1, bd1, Wd2, bd2):
    wbr = Wb.transpose(1, 0, 2).reshape(_D, _Z * _D).astype(jnp.bfloat16)
    ones_bd = (jnp.arange(_Z * _D, dtype=jnp.int32)[:, None] // _D
               == jnp.arange(_Z, dtype=jnp.int32)[None, :]).astype(jnp.bfloat16)
    nblk = _P // _BB
    return pl.pallas_call(
        _bilinear_body,
        grid=(nblk,),
        in_specs=[
            pl.BlockSpec((_BB, _D), lambda i: (i, 0)),
            pl.BlockSpec((_BB, _D), lambda i: (i + _P // _BB, 0)),
            pl.BlockSpec((_D, _Z * _D), lambda i: (0, 0)),
            pl.BlockSpec((_Z * _D, _Z), lambda i: (0, 0)),
            pl.BlockSpec((_Z,), lambda i: (0,)),
            pl.BlockSpec((_Z, _Z), lambda i: (0, 0)),
            pl.BlockSpec((_Z,), lambda i: (0,)),
            pl.BlockSpec((_Z, 1), lambda i: (0, 0)),
            pl.BlockSpec((1,), lambda i: (0,)),
        ],
        out_specs=pl.BlockSpec((_BB, 1), lambda i: (i, 0)),
        out_shape=jax.ShapeDtypeStruct((_P, 1), jnp.float32),
    )(fab, fab, wbr, ones_bd, bb, Wd1, bd1, Wd2, bd2)


def kernel(adj_indices, adj_values, features, idx, W0, b0, W1, b1, W2, b2,
           Wb, bb, Wd1, bd1, Wd2, bd2):
    pad_e = _EP - _E
    spread = jnp.arange(pad_e, dtype=jnp.int32)
    src3 = jnp.concatenate([adj_indices[0], spread]).reshape(
        _NW, _NCH, 1, _EC)
    dst3 = jnp.concatenate([adj_indices[1], spread]).reshape(
        _NW, _NCH, 1, _EC)
    vals3 = lax.bitcast_convert_type(
        jnp.pad(adj_values, (0, pad_e)), jnp.int32
    ).reshape(_NW, _NCH, 1, _EC)
    edata = jnp.concatenate([src3, dst3, vals3],
                            axis=2).reshape(_NW * _NCH, 3, _EC)
    feats_p = jnp.pad(features, ((0, _NP - _N), (0, 0)))
    b0r = b0.reshape(1, _D)
    b1r = b1.reshape(1, _D)
    b2r = b2.reshape(1, _D)

    h0 = _matmul(feats_p, W0)
    p0 = _spmm_partials(h0, edata)
    x1, h1 = _fused_layer(p0, b0r, jnp.zeros_like(h0), W1)
    p1 = _spmm_partials(h1, edata)
    x2, h2 = _fused_layer(p1, b1r, x1, W2)
    p2 = _spmm_partials(h2, edata)
    latent_p = _final_layer(p2, b2r, x2)

    idxflat = jnp.concatenate([idx[0], idx[1]])
    fab = _pair_gather(latent_p, idxflat)
    pred = _bilinear_decode(fab, Wb, bb, Wd1, bd1, Wd2, bd2)
    return (pred, latent_p[:_N])


# speedup vs baseline: 1.8521x; 1.0421x over previous
import dataclasses
import functools

import jax
import jax.numpy as jnp
from jax import lax
from jax.experimental import pallas as pl
from jax.experimental.pallas import tpu as pltpu
from jax.experimental.pallas import tpu_sc as plsc

_N = 10000
_NP = 10240
_E = 320000
_D = 128
_Z = 64
_P = 65536
_BLK = 1024
_BB = 512

_NC = 2
_NS = 16
_NW = _NC * _NS
_EC = 128
_NCH = 80
_EP = _NW * _NCH * _EC
_K0 = 80
_K1 = 80
_RW = _NP // _NS
_PW = 2 * _P // _NW

_sc_mesh = plsc.VectorSubcoreMesh(core_axis_name="c", subcore_axis_name="s")

_sc_params = pltpu.CompilerParams()
if "needs_layout_passes" in pltpu.CompilerParams.__dataclass_fields__:
    _sc_params = dataclasses.replace(_sc_params, needs_layout_passes=False)


def _scale_rows(rows, ebuf):
    @pl.loop(0, _EC)
    def _(r):
        ridx = jnp.full((16,), r, jnp.int32)
        vidx = jnp.full((16,), 2, jnp.int32)
        val = plsc.bitcast(plsc.load_gather(ebuf, [vidx, ridx]), jnp.float32)
        for g in range(_D // 16):
            sl = (r, pl.ds(g * 16, 16))
            rows[sl] = rows[sl] * val


def _spmm_body(x_hbm, ed_hbm, out_hbm,
               eb0_v, eb1_v, rows0_v, rows1_v, acc_sh,
               gsem0, gsem1, esem0, esem1):
    c = lax.axis_index("c")
    s = lax.axis_index("s")
    wid = s * _NC + c
    rows = (rows0_v, rows1_v)
    ebuf = (eb0_v, eb1_v)
    gsem = (gsem0, gsem1)
    esem = (esem0, esem1)

    with jax.named_scope("zero_acc"):
        @pl.loop(0, _EC)
        def _(i):
            for g in range(_D // 16):
                rows0_v[i, pl.ds(g * 16, 16)] = jnp.zeros((16,), jnp.float32)

        @pl.loop(0, _RW // _EC)
        def _(k):
            pltpu.sync_copy(rows0_v, acc_sh.at[pl.ds(s * _RW + k * _EC, _EC)])

    base = lax.select(c == 0, s * _K0, _NS * _K0 + s * _K1)
    cnt = lax.select(c == 0, _K0, _K1)

    pltpu.async_copy(ed_hbm.at[base], eb0_v, esem0)
    pltpu.make_async_copy(ed_hbm.at[base], eb0_v, esem0).wait()
    pltpu.async_copy(x_hbm.at[eb0_v.at[0]], rows0_v, gsem0)
    pltpu.async_copy(ed_hbm.at[base + 1], eb1_v, esem1)
    plsc.subcore_barrier()

    with jax.named_scope("edge_loop"):
        @pl.loop(0, cnt, step=2)
        def _(k0):
          for b in range(2):
            k = k0 + b
            pltpu.make_async_copy(x_hbm.at[ebuf[b].at[0]], rows[b],
                                  gsem[b]).wait()

            @pl.when(k + 1 < cnt)
            def _():
                pltpu.make_async_copy(ed_hbm.at[base + k + 1], ebuf[1 - b],
                                      esem[1 - b]).wait()
                pltpu.async_copy(x_hbm.at[ebuf[1 - b].at[0]], rows[1 - b],
                                 gsem[1 - b])

            _scale_rows(rows[b], ebuf[b])
            pltpu.sync_copy(rows[b], acc_sh.at[ebuf[b].at[1]], add=True)

            @pl.when(k + 2 < cnt)
            def _():
                pltpu.async_copy(ed_hbm.at[base + k + 2], ebuf[b], esem[b])

    with jax.named_scope("tail_barrier"):
        plsc.subcore_barrier()
    with jax.named_scope("copy_out"):
        pltpu.sync_copy(acc_sh.at[pl.ds(s * _RW, _RW)],
                        out_hbm.at[c, pl.ds(s * _RW, _RW)])


def _spmm_partials(x, edata):
    k = pl.kernel(
        _spmm_body,
        out_type=jax.ShapeDtypeStruct((_NC, _NP, _D), jnp.float32),
        mesh=_sc_mesh,
        scratch_types=[
            pltpu.VMEM((3, _EC), jnp.int32),
            pltpu.VMEM((3, _EC), jnp.int32),
            pltpu.VMEM((_EC, _D), jnp.float32),
            pltpu.VMEM((_EC, _D), jnp.float32),
            pltpu.VMEM_SHARED((_NP, _D), jnp.float32),
            pltpu.SemaphoreType.DMA,
            pltpu.SemaphoreType.DMA,
            pltpu.SemaphoreType.DMA,
            pltpu.SemaphoreType.DMA,
        ],
        compiler_params=_sc_params,
    )
    return k(x, edata)


def _pairs_body(lat_hbm, idx_hbm, out_hbm, idx_v, rows_v):
    c = lax.axis_index("c")
    s = lax.axis_index("s")
    wid = s * _NC + c
    base = wid * _PW

    @pl.loop(0, _PW // 128)
    def _(k):
        b = base + k * 128
        pltpu.sync_copy(idx_hbm.at[pl.ds(b, 128)], idx_v)
        pltpu.sync_copy(lat_hbm.at[idx_v], rows_v)
        pltpu.sync_copy(rows_v, out_hbm.at[pl.ds(b, 128)])


def _pair_gather(latent_p, idxflat):
    k = pl.kernel(
        _pairs_body,
        out_type=jax.ShapeDtypeStruct((2 * _P, _D), jnp.float32),
        mesh=_sc_mesh,
        scratch_types=[
            pltpu.VMEM((128,), jnp.int32),
            pltpu.VMEM((128, _D), jnp.float32),
        ],
    )
    return k(latent_p, idxflat)


def _mm_body(x_ref, w_ref, o_ref):
    o_ref[...] = jnp.dot(x_ref[...], w_ref[...],
                         preferred_element_type=jnp.float32)


def _matmul(x, w):
    return pl.pallas_call(
        _mm_body,
        grid=(_NP // _BLK,),
        in_specs=[pl.BlockSpec((_BLK, _D), lambda i: (i, 0)),
                  pl.BlockSpec((_D, _D), lambda i: (0, 0))],
        out_specs=pl.BlockSpec((_BLK, _D), lambda i: (i, 0)),
        out_shape=jax.ShapeDtypeStruct((_NP, _D), jnp.float32),
    )(x, w)


def _fuse_body(p0_ref, p1_ref, b_ref, r_ref, w_ref, x_ref, h_ref):
    x = p0_ref[...] + p1_ref[...] + b_ref[...] + r_ref[...]
    x_ref[...] = x
    h_ref[...] = jnp.dot(x, w_ref[...], preferred_element_type=jnp.float32)


def _fuse_body_nores(p0_ref, p1_ref, b_ref, w_ref, x_ref, h_ref):
    x = p0_ref[...] + p1_ref[...] + b_ref[...]
    x_ref[...] = x
    h_ref[...] = jnp.dot(x, w_ref[...], preferred_element_type=jnp.float32)


def _fused_layer(p, b, res, w):
    blk = pl.BlockSpec((_BLK, _D), lambda i: (i, 0))
    specs = [blk, blk, pl.BlockSpec((1, _D), lambda i: (0, 0))]
    args = [p[0], p[1], b]
    if res is None:
        body = _fuse_body_nores
    else:
        body = _fuse_body
        specs.append(blk)
        args.append(res)
    specs.append(pl.BlockSpec((_D, _D), lambda i: (0, 0)))
    args.append(w)
    return pl.pallas_call(
        body,
        grid=(_NP // _BLK,),
        in_specs=specs,
        out_specs=[blk, blk],
        out_shape=[jax.ShapeDtypeStruct((_NP, _D), jnp.float32),
                   jax.ShapeDtypeStruct((_NP, _D), jnp.float32)],
    )(*args)


def _final_body(p0_ref, p1_ref, b_ref, r_ref, o_ref):
    o_ref[...] = p0_ref[...] + p1_ref[...] + b_ref[...] + r_ref[...]


def _final_layer(p, b, res):
    return pl.pallas_call(
        _final_body,
        grid=(_NP // _BLK,),
        in_specs=[pl.BlockSpec((_BLK, _D), lambda i: (i, 0)),
                  pl.BlockSpec((_BLK, _D), lambda i: (i, 0)),
                  pl.BlockSpec((1, _D), lambda i: (0, 0)),
                  pl.BlockSpec((_BLK, _D), lambda i: (i, 0))],
        out_specs=pl.BlockSpec((_BLK, _D), lambda i: (i, 0)),
        out_shape=jax.ShapeDtypeStruct((_NP, _D), jnp.float32),
    )(p[0], p[1], b, res)


def _bilinear_body(fa_ref, fb_ref, wbr_ref, ones_ref, bb_ref, wd1_ref,
                   bd1_ref, wd2_ref, bd2_ref, out_ref):
    fa = fa_ref[...]
    fb = fb_ref[...].astype(jnp.bfloat16)
    t = jnp.dot(fa.astype(jnp.bfloat16), wbr_ref[...],
                preferred_element_type=jnp.float32)
    fbt = jnp.concatenate([fb] * _Z, axis=1)
    g = t.astype(jnp.bfloat16) * fbt
    feat = jnp.dot(g, ones_ref[...], preferred_element_type=jnp.float32)
    feat = feat + bb_ref[...]
    feat = jnp.where(feat > 0, feat, jnp.exp(jnp.minimum(feat, 0.0)) - 1.0)
    h = jnp.dot(feat, wd1_ref[...], preferred_element_type=jnp.float32)
    h = h + bd1_ref[...]
    h = jnp.where(h > 0, h, jnp.exp(jnp.minimum(h, 0.0)) - 1.0)
    pred = jnp.dot(h, wd2_ref[...], preferred_element_type=jnp.float32)
    out_ref[...] = pred + bd2_ref[...]


def _bilinear_decode(fab, Wb, bb, Wd1, bd1, Wd2, bd2):
    wbr = Wb.transpose(1, 0, 2).reshape(_D, _Z * _D).astype(jnp.bfloat16)
    ones_bd = (jnp.arange(_Z * _D, dtype=jnp.int32)[:, None] // _D
               == jnp.arange(_Z, dtype=jnp.int32)[None, :]).astype(jnp.bfloat16)
    nblk = _P // _BB
    return pl.pallas_call(
        _bilinear_body,
        grid=(nblk,),
        in_specs=[
            pl.BlockSpec((_BB, _D), lambda i: (i, 0)),
            pl.BlockSpec((_BB, _D), lambda i: (i + _P // _BB, 0)),
            pl.BlockSpec((_D, _Z * _D), lambda i: (0, 0)),
            pl.BlockSpec((_Z * _D, _Z), lambda i: (0, 0)),
            pl.BlockSpec((_Z,), lambda i: (0,)),
            pl.BlockSpec((_Z, _Z), lambda i: (0, 0)),
            pl.BlockSpec((_Z,), lambda i: (0,)),
            pl.BlockSpec((_Z, 1), lambda i: (0, 0)),
            pl.BlockSpec((1,), lambda i: (0,)),
        ],
        out_specs=pl.BlockSpec((_BB, 1), lambda i: (i, 0)),
        out_shape=jax.ShapeDtypeStruct((_P, 1), jnp.float32),
    )(fab, fab, wbr, ones_bd, bb, Wd1, bd1, Wd2, bd2)


def kernel(adj_indices, adj_values, features, idx, W0, b0, W1, b1, W2, b2,
           Wb, bb, Wd1, bd1, Wd2, bd2):
    pad_e = _EP - _E
    spread = jnp.arange(pad_e, dtype=jnp.int32)
    src3 = jnp.concatenate([adj_indices[0], spread]).reshape(
        _NW, _NCH, 1, _EC)
    dst3 = jnp.concatenate([adj_indices[1], spread]).reshape(
        _NW, _NCH, 1, _EC)
    vals3 = lax.bitcast_convert_type(
        jnp.pad(adj_values, (0, pad_e)), jnp.int32
    ).reshape(_NW, _NCH, 1, _EC)
    edata = jnp.concatenate([src3, dst3, vals3],
                            axis=2).reshape(_NW * _NCH, 3, _EC)
    feats_p = jnp.pad(features, ((0, _NP - _N), (0, 0)))
    b0r = b0.reshape(1, _D)
    b1r = b1.reshape(1, _D)
    b2r = b2.reshape(1, _D)

    h0 = _matmul(feats_p, W0)
    p0 = _spmm_partials(h0, edata)
    x1, h1 = _fused_layer(p0, b0r, None, W1)
    p1 = _spmm_partials(h1, edata)
    x2, h2 = _fused_layer(p1, b1r, x1, W2)
    p2 = _spmm_partials(h2, edata)
    latent_p = _final_layer(p2, b2r, x2)

    idxflat = jnp.concatenate([idx[0], idx[1]])
    fab = _pair_gather(latent_p, idxflat)
    pred = _bilinear_decode(fab, Wb, bb, Wd1, bd1, Wd2, bd2)
    return (pred, latent_p[:_N])
